# baseline (speedup 1.0000x reference)
import functools

import jax
import jax.numpy as jnp
from jax import lax
from jax.experimental import pallas as pl
from jax.experimental.pallas import tpu as pltpu
from jax.experimental.pallas import tpu_sc as plsc

HIDDEN = 128
N_SUBCORES = 16

_DN = (((1,), (1,)), ((), ()))
_PREC = jax.lax.Precision.HIGHEST


def _embed_body(x_ref, w1_ref, xe_ref, mu_ref):
    xe = lax.dot_general(x_ref[...], w1_ref[...], _DN,
                         preferred_element_type=jnp.float32, precision=_PREC)
    xe_ref[...] = xe
    mu_ref[...] = jnp.maximum(xe, 0.0)


def _iter_body(xe_ref, mi_ref, mo_ref, w2_ref, w3_ref, mu_ref):
    t = lax.dot_general(mi_ref[...], w2_ref[...], _DN,
                        preferred_element_type=jnp.float32, precision=_PREC)
    t = t + lax.dot_general(mo_ref[...], w3_ref[...], _DN,
                            preferred_element_type=jnp.float32, precision=_PREC)
    mu_ref[...] = jnp.maximum(xe_ref[...] + t, 0.0)


def _final_body(xe_ref, mi_ref, mo_ref, w2_ref, w3_ref, h_ref):
    t = lax.dot_general(mi_ref[...], w2_ref[...], _DN,
                        preferred_element_type=jnp.float32, precision=_PREC)
    t = t + lax.dot_general(mo_ref[...], w3_ref[...], _DN,
                            preferred_element_type=jnp.float32, precision=_PREC)
    mu = jnp.maximum(xe_ref[...] + t, 0.0)
    h_ref[...] = jnp.sum(mu, axis=0, keepdims=True)


@functools.cache
def _make_sc_msgs(n_nodes, n_edges):
    per_sub = n_edges // N_SUBCORES
    assert per_sub * N_SUBCORES == n_edges
    win = 0
    for w in range(128, 7, -8):
        if per_sub % w == 0:
            win = w
            break
    assert win > 0
    n_win = per_sub // win
    rows_per_sub = (n_nodes // (8 * N_SUBCORES)) * 8
    rows_rem = n_nodes - rows_per_sub * N_SUBCORES
    assert rows_rem % 8 == 0

    mesh = plsc.VectorSubcoreMesh(core_axis_name="c", subcore_axis_name="s")
    out = jax.ShapeDtypeStruct((n_nodes, HIDDEN), jnp.float32)
    NROW = 4
    NIDX = 8

    @functools.partial(
        pl.kernel,
        out_type=[out, out],
        mesh=mesh,
        scratch_types=[
            pltpu.VMEM((NIDX, 2, win), jnp.int32),
            pltpu.VMEM((NROW, win, HIDDEN), jnp.float32),
            pltpu.VMEM_SHARED((n_nodes, HIDDEN), jnp.float32),
            pltpu.SemaphoreType.DMA((NIDX,)),
            pltpu.SemaphoreType.DMA((NROW,)),
            pltpu.SemaphoreType.DMA((NROW,)),
        ],
    )
    def sc_msgs(mu_hbm, pk_hbm, zeros_hbm, min_hbm, mout_hbm,
                idx_v, rows_v, acc_sh, sem_i, sem_g, sem_s):
        cid = lax.axis_index("c")
        sid = lax.axis_index("s")
        row0 = sid * rows_per_sub
        rem0 = N_SUBCORES * rows_per_sub

        pltpu.sync_copy(zeros_hbm.at[pl.ds(row0, rows_per_sub)],
                        acc_sh.at[pl.ds(row0, rows_per_sub)])
        if rows_rem:
            @pl.when(sid == N_SUBCORES - 1)
            def _():
                pltpu.sync_copy(zeros_hbm.at[pl.ds(rem0, rows_rem)],
                                acc_sh.at[pl.ds(rem0, rows_rem)])
        plsc.subcore_barrier()

        def direction(g, s):
            def idx_copy(k):
                return pltpu.make_async_copy(pk_hbm.at[sid, k],
                                             idx_v.at[k % NIDX],
                                             sem_i.at[k % NIDX])

            def gather_copy(k):
                return pltpu.make_async_copy(mu_hbm.at[idx_v.at[k % NIDX, g]],
                                             rows_v.at[k % NROW],
                                             sem_g.at[k % NROW])

            def scatter_copy(k):
                return pltpu.make_async_copy(rows_v.at[k % NROW],
                                             acc_sh.at[idx_v.at[k % NIDX, s]],
                                             sem_s.at[k % NROW])

            for k in range(5):
                idx_copy(k).start()
            idx_copy(0).wait()
            gather_copy(0).start()

            @pl.loop(0, n_win)
            def _(wi):
                @pl.when(wi >= 3)
                def _():
                    scatter_copy(wi - 3).wait()

                @pl.when(wi + 5 < n_win)
                def _():
                    idx_copy(wi + 5).start()

                @pl.when(wi + 1 < n_win)
                def _():
                    idx_copy(wi + 1).wait()
                    gather_copy(wi + 1).start()

                gather_copy(wi).wait()
                scatter_copy(wi).start(add=True)

            scatter_copy(n_win - 3).wait()
            scatter_copy(n_win - 2).wait()
            scatter_copy(n_win - 1).wait()

        @pl.when(cid == 0)
        def _():
            direction(0, 1)

        @pl.when(cid == 1)
        def _():
            direction(1, 0)

        plsc.subcore_barrier()

        def write_out(o_hbm):
            pltpu.sync_copy(acc_sh.at[pl.ds(row0, rows_per_sub)],
                            o_hbm.at[pl.ds(row0, rows_per_sub)])
            if rows_rem:
                @pl.when(sid == N_SUBCORES - 1)
                def _():
                    pltpu.sync_copy(acc_sh.at[pl.ds(rem0, rows_rem)],
                                    o_hbm.at[pl.ds(rem0, rows_rem)])

        @pl.when(cid == 0)
        def _():
            write_out(min_hbm)

        @pl.when(cid == 1)
        def _():
            write_out(mout_hbm)

    return sc_msgs


@jax.jit
def kernel(x_log, edge_index, W1, W2, W3):
    n_nodes, d_in = x_log.shape
    n_edges = edge_index.shape[1]
    ei = edge_index.astype(jnp.int32)
    src, dst = ei[0], ei[1]
    per_sub = n_edges // N_SUBCORES
    win = 0
    for w in range(128, 7, -8):
        if per_sub % w == 0:
            win = w
            break
    n_win = per_sub // win
    pk = ei.reshape(2, N_SUBCORES, n_win, win).transpose(1, 2, 0, 3)
    zeros = jnp.zeros((n_nodes, HIDDEN), jnp.float32)

    node_mat = jax.ShapeDtypeStruct((n_nodes, HIDDEN), jnp.float32)

    xe, mu = pl.pallas_call(
        _embed_body,
        out_shape=[node_mat, node_mat],
    )(x_log, W1)

    sc_msgs = _make_sc_msgs(n_nodes, n_edges)

    m_in, m_out = sc_msgs(mu, pk, zeros)
    mu = pl.pallas_call(
        _iter_body,
        out_shape=node_mat,
    )(xe, m_in, m_out, W2, W3)

    m_in, m_out = sc_msgs(mu, pk, zeros)
    h = pl.pallas_call(
        _final_body,
        out_shape=jax.ShapeDtypeStruct((1, HIDDEN), jnp.float32),
    )(xe, m_in, m_out, W2, W3)

    return h.reshape((HIDDEN,))


# speedup vs baseline: 1.1431x; 1.1431x over previous
import functools

import jax
import jax.numpy as jnp
from jax import lax
from jax.experimental import pallas as pl
from jax.experimental.pallas import tpu as pltpu
from jax.experimental.pallas import tpu_sc as plsc

HIDDEN = 128
N_SUBCORES = 16

_DN = (((1,), (1,)), ((), ()))
_PREC = jax.lax.Precision.HIGHEST


def _embed_body(x_ref, w1_ref, xe_ref, mu_ref):
    xe = lax.dot_general(x_ref[...], w1_ref[...], _DN,
                         preferred_element_type=jnp.float32, precision=_PREC)
    xe_ref[...] = xe
    mu_ref[...] = jnp.maximum(xe, 0.0)


def _iter_body(xe_ref, mi_ref, mo_ref, w2_ref, w3_ref, mu_ref):
    t = lax.dot_general(mi_ref[...], w2_ref[...], _DN,
                        preferred_element_type=jnp.float32, precision=_PREC)
    t = t + lax.dot_general(mo_ref[...], w3_ref[...], _DN,
                            preferred_element_type=jnp.float32, precision=_PREC)
    mu_ref[...] = jnp.maximum(xe_ref[...] + t, 0.0)


def _final_body(xe_ref, mi_ref, mo_ref, w2_ref, w3_ref, h_ref):
    t = lax.dot_general(mi_ref[...], w2_ref[...], _DN,
                        preferred_element_type=jnp.float32, precision=_PREC)
    t = t + lax.dot_general(mo_ref[...], w3_ref[...], _DN,
                            preferred_element_type=jnp.float32, precision=_PREC)
    mu = jnp.maximum(xe_ref[...] + t, 0.0)
    h_ref[...] = jnp.sum(mu, axis=0, keepdims=True)


@functools.cache
def _make_sc_msgs(n_nodes, n_edges):
    per_sub = n_edges // N_SUBCORES
    assert per_sub * N_SUBCORES == n_edges
    win = 0
    for w in range(128, 7, -8):
        if per_sub % w == 0:
            win = w
            break
    assert win > 0
    n_win = per_sub // win
    rows_per_sub = (n_nodes // (8 * N_SUBCORES)) * 8
    rows_rem = n_nodes - rows_per_sub * N_SUBCORES
    assert rows_rem % 8 == 0

    mesh = plsc.VectorSubcoreMesh(core_axis_name="c", subcore_axis_name="s")
    out = jax.ShapeDtypeStruct((n_nodes, HIDDEN), jnp.float32)
    NROW = 4
    NIDX = 8

    @functools.partial(
        pl.kernel,
        out_type=[out, out],
        mesh=mesh,
        scratch_types=[
            pltpu.VMEM((NIDX, 2, win), jnp.int32),
            pltpu.VMEM((NROW, win, HIDDEN), jnp.float32),
            pltpu.VMEM_SHARED((n_nodes, HIDDEN), jnp.float32),
            pltpu.SemaphoreType.DMA((NIDX,)),
            pltpu.SemaphoreType.DMA((NROW,)),
            pltpu.SemaphoreType.DMA((NROW,)),
        ],
    )
    def sc_msgs(mu_hbm, pk_hbm, min_hbm, mout_hbm,
                idx_v, rows_v, acc_sh, sem_i, sem_g, sem_s):
        cid = lax.axis_index("c")
        sid = lax.axis_index("s")
        row0 = sid * rows_per_sub
        rem0 = N_SUBCORES * rows_per_sub

        z = jnp.zeros((16,), jnp.float32)

        @pl.loop(0, win)
        def _(r):
            @pl.loop(0, HIDDEN, step=16)
            def _(c):
                rows_v[0, r, pl.ds(c, 16)] = z

        n_blk = rows_per_sub // win
        blk_rem = rows_per_sub - n_blk * win

        @pl.loop(0, n_blk)
        def _(j):
            pltpu.sync_copy(rows_v.at[0],
                            acc_sh.at[pl.ds(row0 + j * win, win)])
        if blk_rem:
            pltpu.sync_copy(rows_v.at[0, pl.ds(0, blk_rem)],
                            acc_sh.at[pl.ds(row0 + n_blk * win, blk_rem)])
        if rows_rem:
            @pl.when(sid == N_SUBCORES - 1)
            def _():
                @pl.loop(0, rows_rem // 16)
                def _(j):
                    pltpu.sync_copy(
                        rows_v.at[0, pl.ds(0, 16)],
                        acc_sh.at[pl.ds(rem0 + j * 16, 16)])
        plsc.subcore_barrier()

        def direction(g, s):
            def idx_copy(k):
                return pltpu.make_async_copy(pk_hbm.at[sid, k],
                                             idx_v.at[k % NIDX],
                                             sem_i.at[k % NIDX])

            def gather_copy(k):
                return pltpu.make_async_copy(mu_hbm.at[idx_v.at[k % NIDX, g]],
                                             rows_v.at[k % NROW],
                                             sem_g.at[k % NROW])

            def scatter_copy(k):
                return pltpu.make_async_copy(rows_v.at[k % NROW],
                                             acc_sh.at[idx_v.at[k % NIDX, s]],
                                             sem_s.at[k % NROW])

            for k in range(6):
                idx_copy(k).start()
            for k in range(2):
                idx_copy(k).wait()
                gather_copy(k).start()

            @pl.loop(0, n_win)
            def _(wi):
                @pl.when(wi >= 2)
                def _():
                    scatter_copy(wi - 2).wait()

                @pl.when(wi + 6 < n_win)
                def _():
                    idx_copy(wi + 6).start()

                @pl.when(wi + 2 < n_win)
                def _():
                    idx_copy(wi + 2).wait()
                    gather_copy(wi + 2).start()

                gather_copy(wi).wait()
                scatter_copy(wi).start(add=True)

            scatter_copy(n_win - 2).wait()
            scatter_copy(n_win - 1).wait()

        @pl.when(cid == 0)
        def _():
            direction(0, 1)

        @pl.when(cid == 1)
        def _():
            direction(1, 0)

        plsc.subcore_barrier()

        def write_out(o_hbm):
            pltpu.sync_copy(acc_sh.at[pl.ds(row0, rows_per_sub)],
                            o_hbm.at[pl.ds(row0, rows_per_sub)])
            if rows_rem:
                @pl.when(sid == N_SUBCORES - 1)
                def _():
                    pltpu.sync_copy(acc_sh.at[pl.ds(rem0, rows_rem)],
                                    o_hbm.at[pl.ds(rem0, rows_rem)])

        @pl.when(cid == 0)
        def _():
            write_out(min_hbm)

        @pl.when(cid == 1)
        def _():
            write_out(mout_hbm)

    return sc_msgs


@jax.jit
def kernel(x_log, edge_index, W1, W2, W3):
    n_nodes, d_in = x_log.shape
    n_edges = edge_index.shape[1]
    ei = edge_index.astype(jnp.int32)
    src, dst = ei[0], ei[1]
    per_sub = n_edges // N_SUBCORES
    win = 0
    for w in range(128, 7, -8):
        if per_sub % w == 0:
            win = w
            break
    n_win = per_sub // win
    pk = ei.reshape(2, N_SUBCORES, n_win, win).transpose(1, 2, 0, 3)

    node_mat = jax.ShapeDtypeStruct((n_nodes, HIDDEN), jnp.float32)

    xe, mu = pl.pallas_call(
        _embed_body,
        out_shape=[node_mat, node_mat],
    )(x_log, W1)

    sc_msgs = _make_sc_msgs(n_nodes, n_edges)

    m_in, m_out = sc_msgs(mu, pk)
    mu = pl.pallas_call(
        _iter_body,
        out_shape=node_mat,
    )(xe, m_in, m_out, W2, W3)

    m_in, m_out = sc_msgs(mu, pk)
    h = pl.pallas_call(
        _final_body,
        out_shape=jax.ShapeDtypeStruct((1, HIDDEN), jnp.float32),
    )(xe, m_in, m_out, W2, W3)

    return h.reshape((HIDDEN,))


# speedup vs baseline: 1.1471x; 1.0035x over previous
import functools

import jax
import jax.numpy as jnp
from jax import lax
from jax.experimental import pallas as pl
from jax.experimental.pallas import tpu as pltpu
from jax.experimental.pallas import tpu_sc as plsc

HIDDEN = 128
N_SUBCORES = 16

_DN = (((1,), (1,)), ((), ()))
_PREC = jax.lax.Precision.HIGHEST


def _embed_body(x_ref, w1_ref, xe_ref, mu_ref):
    xe = lax.dot_general(x_ref[...], w1_ref[...], _DN,
                         preferred_element_type=jnp.float32, precision=_PREC)
    xe_ref[...] = xe
    mu_ref[...] = jnp.maximum(xe, 0.0)


def _iter_body(xe_ref, mi_ref, mo_ref, w2_ref, w3_ref, mu_ref):
    t = lax.dot_general(mi_ref[...], w2_ref[...], _DN,
                        preferred_element_type=jnp.float32, precision=_PREC)
    t = t + lax.dot_general(mo_ref[...], w3_ref[...], _DN,
                            preferred_element_type=jnp.float32, precision=_PREC)
    mu_ref[...] = jnp.maximum(xe_ref[...] + t, 0.0)


def _final_body(xe_ref, mi_ref, mo_ref, w2_ref, w3_ref, h_ref):
    t = lax.dot_general(mi_ref[...], w2_ref[...], _DN,
                        preferred_element_type=jnp.float32, precision=_PREC)
    t = t + lax.dot_general(mo_ref[...], w3_ref[...], _DN,
                            preferred_element_type=jnp.float32, precision=_PREC)
    mu = jnp.maximum(xe_ref[...] + t, 0.0)
    h_ref[...] = jnp.sum(mu, axis=0, keepdims=True)


@functools.cache
def _make_sc_msgs(n_nodes, n_edges):
    per_sub = n_edges // N_SUBCORES
    assert per_sub * N_SUBCORES == n_edges
    win = 0
    for w in range(128, 7, -8):
        if per_sub % w == 0:
            win = w
            break
    assert win > 0
    n_win = per_sub // win
    rows_per_sub = (n_nodes // (8 * N_SUBCORES)) * 8
    rows_rem = n_nodes - rows_per_sub * N_SUBCORES
    assert rows_rem % 8 == 0

    mesh = plsc.VectorSubcoreMesh(core_axis_name="c", subcore_axis_name="s")
    out = jax.ShapeDtypeStruct((n_nodes, HIDDEN), jnp.float32)
    NROW = 4
    NIDX = 8

    @functools.partial(
        pl.kernel,
        out_type=[out, out],
        mesh=mesh,
        scratch_types=[
            pltpu.VMEM((NIDX, 2, win), jnp.int32),
            pltpu.VMEM((NROW, win, HIDDEN), jnp.float32),
            pltpu.VMEM_SHARED((n_nodes, HIDDEN), jnp.float32),
            pltpu.SemaphoreType.DMA((NIDX,)),
            pltpu.SemaphoreType.DMA((NROW,)),
            pltpu.SemaphoreType.DMA((NROW,)),
        ],
    )
    def sc_msgs(mu_hbm, pk_hbm, min_hbm, mout_hbm,
                idx_v, rows_v, acc_sh, sem_i, sem_g, sem_s):
        cid = lax.axis_index("c")
        sid = lax.axis_index("s")
        row0 = sid * rows_per_sub
        rem0 = N_SUBCORES * rows_per_sub

        z = jnp.zeros((16,), jnp.float32)

        @pl.loop(0, win)
        def _(r):
            @pl.loop(0, HIDDEN, step=16)
            def _(c):
                rows_v[0, r, pl.ds(c, 16)] = z

        n_blk = rows_per_sub // win
        blk_rem = rows_per_sub - n_blk * win

        def zero_copy(j):
            return pltpu.make_async_copy(
                rows_v.at[0], acc_sh.at[pl.ds(row0 + j * win, win)],
                sem_g.at[0])

        for j in range(n_blk):
            zero_copy(j).start()
        if blk_rem:
            pltpu.async_copy(rows_v.at[0, pl.ds(0, blk_rem)],
                             acc_sh.at[pl.ds(row0 + n_blk * win, blk_rem)],
                             sem_g.at[1])
        if rows_rem:
            @pl.when(sid == N_SUBCORES - 1)
            def _():
                pltpu.async_copy(rows_v.at[0, pl.ds(0, rows_rem)],
                                 acc_sh.at[pl.ds(rem0, rows_rem)],
                                 sem_g.at[2])
        for j in range(n_blk):
            zero_copy(j).wait()
        if blk_rem:
            pltpu.make_async_copy(
                rows_v.at[0, pl.ds(0, blk_rem)],
                acc_sh.at[pl.ds(row0 + n_blk * win, blk_rem)],
                sem_g.at[1]).wait()
        if rows_rem:
            @pl.when(sid == N_SUBCORES - 1)
            def _():
                pltpu.make_async_copy(
                    rows_v.at[0, pl.ds(0, rows_rem)],
                    acc_sh.at[pl.ds(rem0, rows_rem)],
                    sem_g.at[2]).wait()
        plsc.subcore_barrier()

        def direction(g, s):
            def idx_copy(k):
                return pltpu.make_async_copy(pk_hbm.at[sid, k],
                                             idx_v.at[k % NIDX],
                                             sem_i.at[k % NIDX])

            def gather_copy(k):
                return pltpu.make_async_copy(mu_hbm.at[idx_v.at[k % NIDX, g]],
                                             rows_v.at[k % NROW],
                                             sem_g.at[k % NROW])

            def scatter_copy(k):
                return pltpu.make_async_copy(rows_v.at[k % NROW],
                                             acc_sh.at[idx_v.at[k % NIDX, s]],
                                             sem_s.at[k % NROW])

            for k in range(6):
                idx_copy(k).start()
            for k in range(2):
                idx_copy(k).wait()
                gather_copy(k).start()

            @pl.loop(0, n_win)
            def _(wi):
                @pl.when(wi >= 2)
                def _():
                    scatter_copy(wi - 2).wait()

                @pl.when(wi + 6 < n_win)
                def _():
                    idx_copy(wi + 6).start()

                @pl.when(wi + 2 < n_win)
                def _():
                    idx_copy(wi + 2).wait()
                    gather_copy(wi + 2).start()

                gather_copy(wi).wait()
                scatter_copy(wi).start(add=True)

            scatter_copy(n_win - 2).wait()
            scatter_copy(n_win - 1).wait()

        @pl.when(cid == 0)
        def _():
            direction(0, 1)

        @pl.when(cid == 1)
        def _():
            direction(1, 0)

        plsc.subcore_barrier()

        def write_out(o_hbm):
            pltpu.sync_copy(acc_sh.at[pl.ds(row0, rows_per_sub)],
                            o_hbm.at[pl.ds(row0, rows_per_sub)])
            if rows_rem:
                @pl.when(sid == N_SUBCORES - 1)
                def _():
                    pltpu.sync_copy(acc_sh.at[pl.ds(rem0, rows_rem)],
                                    o_hbm.at[pl.ds(rem0, rows_rem)])

        @pl.when(cid == 0)
        def _():
            write_out(min_hbm)

        @pl.when(cid == 1)
        def _():
            write_out(mout_hbm)

    return sc_msgs


@jax.jit
def kernel(x_log, edge_index, W1, W2, W3):
    n_nodes, d_in = x_log.shape
    n_edges = edge_index.shape[1]
    ei = edge_index.astype(jnp.int32)
    src, dst = ei[0], ei[1]
    per_sub = n_edges // N_SUBCORES
    win = 0
    for w in range(128, 7, -8):
        if per_sub % w == 0:
            win = w
            break
    n_win = per_sub // win
    pk = ei.reshape(2, N_SUBCORES, n_win, win).transpose(1, 2, 0, 3)

    node_mat = jax.ShapeDtypeStruct((n_nodes, HIDDEN), jnp.float32)

    xe, mu = pl.pallas_call(
        _embed_body,
        out_shape=[node_mat, node_mat],
    )(x_log, W1)

    sc_msgs = _make_sc_msgs(n_nodes, n_edges)

    m_in, m_out = sc_msgs(mu, pk)
    mu = pl.pallas_call(
        _iter_body,
        out_shape=node_mat,
    )(xe, m_in, m_out, W2, W3)

    m_in, m_out = sc_msgs(mu, pk)
    h = pl.pallas_call(
        _final_body,
        out_shape=jax.ShapeDtypeStruct((1, HIDDEN), jnp.float32),
    )(xe, m_in, m_out, W2, W3)

    return h.reshape((HIDDEN,))


# speedup vs baseline: 1.1990x; 1.0453x over previous
import functools

import jax
import jax.numpy as jnp
from jax import lax
from jax.experimental import pallas as pl
from jax.experimental.pallas import tpu as pltpu
from jax.experimental.pallas import tpu_sc as plsc

HIDDEN = 128
N_SUBCORES = 16

_DN = (((1,), (1,)), ((), ()))
_PREC = jax.lax.Precision.DEFAULT


def _embed_body(x_ref, w1_ref, xe_ref, mu_ref):
    xe = lax.dot_general(x_ref[...], w1_ref[...], _DN,
                         preferred_element_type=jnp.float32, precision=_PREC)
    xe_ref[...] = xe
    mu_ref[...] = jnp.maximum(xe, 0.0)


def _iter_body(xe_ref, mi_ref, mo_ref, w2_ref, w3_ref, mu_ref):
    t = lax.dot_general(mi_ref[...], w2_ref[...], _DN,
                        preferred_element_type=jnp.float32, precision=_PREC)
    t = t + lax.dot_general(mo_ref[...], w3_ref[...], _DN,
                            preferred_element_type=jnp.float32, precision=_PREC)
    mu_ref[...] = jnp.maximum(xe_ref[...] + t, 0.0)


def _final_body(xe_ref, mi_ref, mo_ref, w2_ref, w3_ref, h_ref):
    t = lax.dot_general(mi_ref[...], w2_ref[...], _DN,
                        preferred_element_type=jnp.float32, precision=_PREC)
    t = t + lax.dot_general(mo_ref[...], w3_ref[...], _DN,
                            preferred_element_type=jnp.float32, precision=_PREC)
    mu = jnp.maximum(xe_ref[...] + t, 0.0)
    h_ref[...] = jnp.sum(mu, axis=0, keepdims=True)


@functools.cache
def _make_sc_msgs(n_nodes, n_edges):
    per_sub = n_edges // N_SUBCORES
    assert per_sub * N_SUBCORES == n_edges
    win = 0
    for w in range(128, 7, -8):
        if per_sub % w == 0:
            win = w
            break
    assert win > 0
    n_win = per_sub // win
    rows_per_sub = (n_nodes // (8 * N_SUBCORES)) * 8
    rows_rem = n_nodes - rows_per_sub * N_SUBCORES
    assert rows_rem % 8 == 0

    mesh = plsc.VectorSubcoreMesh(core_axis_name="c", subcore_axis_name="s")
    out = jax.ShapeDtypeStruct((n_nodes, HIDDEN), jnp.float32)
    NROW = 4
    NIDX = 8

    @functools.partial(
        pl.kernel,
        out_type=[out, out],
        mesh=mesh,
        scratch_types=[
            pltpu.VMEM((NIDX, 2, win), jnp.int32),
            pltpu.VMEM((NROW, win, HIDDEN), jnp.float32),
            pltpu.VMEM_SHARED((n_nodes, HIDDEN), jnp.float32),
            pltpu.SemaphoreType.DMA((NIDX,)),
            pltpu.SemaphoreType.DMA((NROW,)),
            pltpu.SemaphoreType.DMA((NROW,)),
        ],
    )
    def sc_msgs(mu_hbm, pk_hbm, min_hbm, mout_hbm,
                idx_v, rows_v, acc_sh, sem_i, sem_g, sem_s):
        cid = lax.axis_index("c")
        sid = lax.axis_index("s")
        row0 = sid * rows_per_sub
        rem0 = N_SUBCORES * rows_per_sub

        z = jnp.zeros((16,), jnp.float32)

        @pl.loop(0, win)
        def _(r):
            @pl.loop(0, HIDDEN, step=16)
            def _(c):
                rows_v[0, r, pl.ds(c, 16)] = z

        n_blk = rows_per_sub // win
        blk_rem = rows_per_sub - n_blk * win

        def zero_copy(j):
            return pltpu.make_async_copy(
                rows_v.at[0], acc_sh.at[pl.ds(row0 + j * win, win)],
                sem_g.at[0])

        for j in range(n_blk):
            zero_copy(j).start()
        if blk_rem:
            pltpu.async_copy(rows_v.at[0, pl.ds(0, blk_rem)],
                             acc_sh.at[pl.ds(row0 + n_blk * win, blk_rem)],
                             sem_g.at[1])
        if rows_rem:
            @pl.when(sid == N_SUBCORES - 1)
            def _():
                pltpu.async_copy(rows_v.at[0, pl.ds(0, rows_rem)],
                                 acc_sh.at[pl.ds(rem0, rows_rem)],
                                 sem_g.at[2])
        for j in range(n_blk):
            zero_copy(j).wait()
        if blk_rem:
            pltpu.make_async_copy(
                rows_v.at[0, pl.ds(0, blk_rem)],
                acc_sh.at[pl.ds(row0 + n_blk * win, blk_rem)],
                sem_g.at[1]).wait()
        if rows_rem:
            @pl.when(sid == N_SUBCORES - 1)
            def _():
                pltpu.make_async_copy(
                    rows_v.at[0, pl.ds(0, rows_rem)],
                    acc_sh.at[pl.ds(rem0, rows_rem)],
                    sem_g.at[2]).wait()
        plsc.subcore_barrier()

        def direction(g, s):
            def idx_copy(k):
                return pltpu.make_async_copy(pk_hbm.at[sid, k],
                                             idx_v.at[k % NIDX],
                                             sem_i.at[k % NIDX])

            def gather_copy(k):
                return pltpu.make_async_copy(mu_hbm.at[idx_v.at[k % NIDX, g]],
                                             rows_v.at[k % NROW],
                                             sem_g.at[k % NROW])

            def scatter_copy(k):
                return pltpu.make_async_copy(rows_v.at[k % NROW],
                                             acc_sh.at[idx_v.at[k % NIDX, s]],
                                             sem_s.at[k % NROW])

            for k in range(6):
                idx_copy(k).start()
            for k in range(2):
                idx_copy(k).wait()
                gather_copy(k).start()

            @pl.loop(0, n_win)
            def _(wi):
                @pl.when(wi >= 2)
                def _():
                    scatter_copy(wi - 2).wait()

                @pl.when(wi + 6 < n_win)
                def _():
                    idx_copy(wi + 6).start()

                @pl.when(wi + 2 < n_win)
                def _():
                    idx_copy(wi + 2).wait()
                    gather_copy(wi + 2).start()

                gather_copy(wi).wait()
                scatter_copy(wi).start(add=True)

            scatter_copy(n_win - 2).wait()
            scatter_copy(n_win - 1).wait()

        @pl.when(cid == 0)
        def _():
            direction(0, 1)

        @pl.when(cid == 1)
        def _():
            direction(1, 0)

        plsc.subcore_barrier()

        def write_out(o_hbm):
            pltpu.sync_copy(acc_sh.at[pl.ds(row0, rows_per_sub)],
                            o_hbm.at[pl.ds(row0, rows_per_sub)])
            if rows_rem:
                @pl.when(sid == N_SUBCORES - 1)
                def _():
                    pltpu.sync_copy(acc_sh.at[pl.ds(rem0, rows_rem)],
                                    o_hbm.at[pl.ds(rem0, rows_rem)])

        @pl.when(cid == 0)
        def _():
            write_out(min_hbm)

        @pl.when(cid == 1)
        def _():
            write_out(mout_hbm)

    return sc_msgs


@jax.jit
def kernel(x_log, edge_index, W1, W2, W3):
    n_nodes, d_in = x_log.shape
    n_edges = edge_index.shape[1]
    ei = edge_index.astype(jnp.int32)
    src, dst = ei[0], ei[1]
    per_sub = n_edges // N_SUBCORES
    win = 0
    for w in range(128, 7, -8):
        if per_sub % w == 0:
            win = w
            break
    n_win = per_sub // win
    pk = ei.reshape(2, N_SUBCORES, n_win, win).transpose(1, 2, 0, 3)

    node_mat = jax.ShapeDtypeStruct((n_nodes, HIDDEN), jnp.float32)

    xe, mu = pl.pallas_call(
        _embed_body,
        out_shape=[node_mat, node_mat],
    )(x_log, W1)

    sc_msgs = _make_sc_msgs(n_nodes, n_edges)

    m_in, m_out = sc_msgs(mu, pk)
    mu = pl.pallas_call(
        _iter_body,
        out_shape=node_mat,
    )(xe, m_in, m_out, W2, W3)

    m_in, m_out = sc_msgs(mu, pk)
    h = pl.pallas_call(
        _final_body,
        out_shape=jax.ShapeDtypeStruct((1, HIDDEN), jnp.float32),
    )(xe, m_in, m_out, W2, W3)

    return h.reshape((HIDDEN,))


# speedup vs baseline: 1.2016x; 1.0021x over previous
import functools

import jax
import jax.numpy as jnp
from jax import lax
from jax.experimental import pallas as pl
from jax.experimental.pallas import tpu as pltpu
from jax.experimental.pallas import tpu_sc as plsc

HIDDEN = 128
N_SUBCORES = 16

_DN = (((1,), (1,)), ((), ()))
_PREC = jax.lax.Precision.DEFAULT


def _embed_body(x_ref, w1_ref, xe_ref, mu_ref):
    xe = lax.dot_general(x_ref[...], w1_ref[...], _DN,
                         preferred_element_type=jnp.float32, precision=_PREC)
    xe_ref[...] = xe
    mu_ref[...] = jnp.maximum(xe, 0.0)


def _iter_body(xe_ref, mi_ref, mo_ref, w2_ref, w3_ref, mu_ref):
    t = lax.dot_general(mi_ref[...], w2_ref[...], _DN,
                        preferred_element_type=jnp.float32, precision=_PREC)
    t = t + lax.dot_general(mo_ref[...], w3_ref[...], _DN,
                            preferred_element_type=jnp.float32, precision=_PREC)
    mu_ref[...] = jnp.maximum(xe_ref[...] + t, 0.0)


def _final_body(xe_ref, mi_ref, mo_ref, w2_ref, w3_ref, h_ref):
    t = lax.dot_general(mi_ref[...], w2_ref[...], _DN,
                        preferred_element_type=jnp.float32, precision=_PREC)
    t = t + lax.dot_general(mo_ref[...], w3_ref[...], _DN,
                            preferred_element_type=jnp.float32, precision=_PREC)
    mu = jnp.maximum(xe_ref[...] + t, 0.0)
    h_ref[...] = jnp.sum(mu, axis=0, keepdims=True)


@functools.cache
def _make_sc_msgs(n_nodes, n_edges):
    per_sub = n_edges // N_SUBCORES
    assert per_sub * N_SUBCORES == n_edges
    win = 0
    for w in range(128, 7, -8):
        if per_sub % w == 0:
            win = w
            break
    assert win > 0
    n_win = per_sub // win
    rows_per_sub = (n_nodes // (8 * N_SUBCORES)) * 8
    rows_rem = n_nodes - rows_per_sub * N_SUBCORES
    assert rows_rem % 8 == 0

    mesh = plsc.VectorSubcoreMesh(core_axis_name="c", subcore_axis_name="s")
    out = jax.ShapeDtypeStruct((n_nodes, HIDDEN), jnp.float32)
    NROW = 4
    NIDX = 8

    @functools.partial(
        pl.kernel,
        out_type=[out, out],
        mesh=mesh,
        scratch_types=[
            pltpu.VMEM((NIDX, 2, win), jnp.int32),
            pltpu.VMEM((NROW, win, HIDDEN), jnp.float32),
            pltpu.VMEM_SHARED((n_nodes, HIDDEN), jnp.float32),
            pltpu.SemaphoreType.DMA((NIDX,)),
            pltpu.SemaphoreType.DMA((NROW,)),
            pltpu.SemaphoreType.DMA((NROW,)),
        ],
    )
    def sc_msgs(mu_hbm, pk_hbm, min_hbm, mout_hbm,
                idx_v, rows_v, acc_sh, sem_i, sem_g, sem_s):
        cid = lax.axis_index("c")
        sid = lax.axis_index("s")
        row0 = sid * rows_per_sub
        rem0 = N_SUBCORES * rows_per_sub

        z = jnp.zeros((16,), jnp.float32)

        @pl.loop(0, win)
        def _(r):
            @pl.loop(0, HIDDEN, step=16)
            def _(c):
                rows_v[0, r, pl.ds(c, 16)] = z

        n_blk = rows_per_sub // win
        blk_rem = rows_per_sub - n_blk * win

        def zero_copy(j):
            return pltpu.make_async_copy(
                rows_v.at[0], acc_sh.at[pl.ds(row0 + j * win, win)],
                sem_g.at[0])

        for j in range(n_blk):
            zero_copy(j).start()
        if blk_rem:
            pltpu.async_copy(rows_v.at[0, pl.ds(0, blk_rem)],
                             acc_sh.at[pl.ds(row0 + n_blk * win, blk_rem)],
                             sem_g.at[1])
        if rows_rem:
            @pl.when(sid == N_SUBCORES - 1)
            def _():
                pltpu.async_copy(rows_v.at[0, pl.ds(0, rows_rem)],
                                 acc_sh.at[pl.ds(rem0, rows_rem)],
                                 sem_g.at[2])
        for j in range(n_blk):
            zero_copy(j).wait()
        if blk_rem:
            pltpu.make_async_copy(
                rows_v.at[0, pl.ds(0, blk_rem)],
                acc_sh.at[pl.ds(row0 + n_blk * win, blk_rem)],
                sem_g.at[1]).wait()
        if rows_rem:
            @pl.when(sid == N_SUBCORES - 1)
            def _():
                pltpu.make_async_copy(
                    rows_v.at[0, pl.ds(0, rows_rem)],
                    acc_sh.at[pl.ds(rem0, rows_rem)],
                    sem_g.at[2]).wait()
        plsc.subcore_barrier()

        def direction(g, s):
            def idx_copy(k):
                return pltpu.make_async_copy(pk_hbm.at[sid, k],
                                             idx_v.at[k % NIDX],
                                             sem_i.at[k % NIDX])

            def gather_copy(k):
                return pltpu.make_async_copy(mu_hbm.at[idx_v.at[k % NIDX, g]],
                                             rows_v.at[k % NROW],
                                             sem_g.at[k % NROW])

            def scatter_copy(k):
                return pltpu.make_async_copy(rows_v.at[k % NROW],
                                             acc_sh.at[idx_v.at[k % NIDX, s]],
                                             sem_s.at[k % NROW])

            for k in range(6):
                idx_copy(k).start()
            for k in range(2):
                idx_copy(k).wait()
                gather_copy(k).start()

            @pl.loop(0, n_win)
            def _(wi):
                @pl.when(wi >= 2)
                def _():
                    scatter_copy(wi - 2).wait()

                @pl.when(wi + 6 < n_win)
                def _():
                    idx_copy(wi + 6).start()

                @pl.when(wi + 2 < n_win)
                def _():
                    idx_copy(wi + 2).wait()
                    gather_copy(wi + 2).start()

                gather_copy(wi).wait()
                scatter_copy(wi).start(add=True)

            scatter_copy(n_win - 2).wait()
            scatter_copy(n_win - 1).wait()

        @pl.when(cid == 0)
        def _():
            direction(0, 1)

        @pl.when(cid == 1)
        def _():
            direction(1, 0)

        plsc.subcore_barrier()

        def write_out(o_hbm):
            pltpu.sync_copy(acc_sh.at[pl.ds(row0, rows_per_sub)],
                            o_hbm.at[pl.ds(row0, rows_per_sub)])
            if rows_rem:
                @pl.when(sid == N_SUBCORES - 1)
                def _():
                    pltpu.sync_copy(acc_sh.at[pl.ds(rem0, rows_rem)],
                                    o_hbm.at[pl.ds(rem0, rows_rem)])

        @pl.when(cid == 0)
        def _():
            write_out(min_hbm)

        @pl.when(cid == 1)
        def _():
            write_out(mout_hbm)

    return sc_msgs


@jax.jit
def kernel(x_log, edge_index, W1, W2, W3):
    n_nodes, d_in = x_log.shape
    n_edges = edge_index.shape[1]
    ei = edge_index.astype(jnp.int32)
    per_sub = n_edges // N_SUBCORES
    win = 0
    for w in range(128, 7, -8):
        if per_sub % w == 0:
            win = w
            break
    n_win = per_sub // win
    pk = ei.reshape(2, N_SUBCORES, n_win, win).transpose(1, 2, 0, 3)

    node_mat = jax.ShapeDtypeStruct((n_nodes, HIDDEN), jnp.float32)

    xe, mu = pl.pallas_call(
        _embed_body,
        out_shape=[node_mat, node_mat],
    )(x_log, W1)

    sc_msgs = _make_sc_msgs(n_nodes, n_edges)

    m_in, m_out = sc_msgs(mu, pk)
    mu = pl.pallas_call(
        _iter_body,
        out_shape=node_mat,
    )(xe, m_in, m_out, W2, W3)

    m_in, m_out = sc_msgs(mu, pk)
    h = pl.pallas_call(
        _final_body,
        out_shape=jax.ShapeDtypeStruct((1, HIDDEN), jnp.float32),
    )(xe, m_in, m_out, W2, W3)

    return h.reshape((HIDDEN,))
